# baseline (speedup 1.0000x reference)
import functools

import jax
import jax.numpy as jnp
from jax import lax
from jax.experimental import pallas as pl
from jax.experimental.pallas import tpu as pltpu
from jax.experimental.pallas import tpu_sc as plsc

MF_DIM = 32
BATCH = 16384
NC = 2
NS = 16
NW = NC * NS
B_PER_W = BATCH // NW
SW = 512
NSTRIP = 1954
KMAX = (NSTRIP + NW - 1) // NW
NCHUNKS = BATCH // 16


def _extract_body(ui_hbm, ii_hbm, ut_hbm, it_hbm, eu_hbm, ei_hbm,
                  idx_v, pos_v, strip, stageT, rows_st, sem, sem2):
    c = lax.axis_index("c")
    s_ax = lax.axis_index("s")
    wid = s_ax * NC + c
    lanes = lax.iota(jnp.int32, 16)

    for idx_hbm, tab_hbm, out_hbm in ((ui_hbm, ut_hbm, eu_hbm),
                                      (ii_hbm, it_hbm, ei_hbm)):
        pltpu.sync_copy(idx_hbm, idx_v)

        def sel(i, off):
            r = idx_v[pl.ds(i * 16, 16)]
            m = ((r >> 9) & (NW - 1)) == wid
            plsc.store_compressed(pos_v.at[pl.ds(off, 16)],
                                  i * 16 + lanes, mask=m)
            return off + plsc.all_reduce_population_count(m)[0]

        n = lax.fori_loop(0, NCHUNKS, sel, 0)

        def fire(k):
            s = wid + k * NW

            @pl.when(s < NSTRIP - 1)
            def _():
                base = pl.multiple_of(s * SW, SW)
                pltpu.async_copy(tab_hbm.at[:, pl.ds(base, SW)],
                                 strip.at[k % 2], sem)

            @pl.when(s == NSTRIP - 1)
            def _():
                base = pl.multiple_of(s * SW, SW)
                pltpu.async_copy(tab_hbm.at[:, pl.ds(base, 128)],
                                 strip.at[k % 2, :, pl.ds(0, 128)], sem)

        def drain(k):
            s = wid + k * NW

            @pl.when(s < NSTRIP - 1)
            def _():
                pltpu.make_async_copy(tab_hbm.at[:, pl.ds(0, SW)],
                                      strip.at[k % 2], sem).wait()

            @pl.when(s == NSTRIP - 1)
            def _():
                pltpu.make_async_copy(
                    tab_hbm.at[:, pl.ds(0, 128)],
                    strip.at[k % 2, :, pl.ds(0, 128)], sem).wait()

        fire(0)

        def k_body(k, carry):
            s = wid + k * NW

            @pl.when(s < NSTRIP)
            def _():
                drain(k)

                @pl.when(k + 1 < KMAX)
                def _():
                    fire(k + 1)

                nch = (n + 15) >> 4

                def ch_body(j, carry2):
                    pos = pos_v[pl.ds(j * 16, 16)]
                    pos = jnp.minimum(jnp.maximum(pos, 0), BATCH - 1)
                    lane_ok = (j * 16 + lanes) < n
                    r = plsc.load_gather(idx_v, [pos])
                    m = lane_ok & ((r >> 9) == s)
                    cnt = plsc.all_reduce_population_count(m)[0]

                    @pl.when(cnt > 0)
                    def _():
                        col = jnp.minimum(
                            jnp.maximum(r - s * SW, 0), SW - 1)
                        for d in range(MF_DIM):
                            x_d = plsc.load_gather(
                                strip.at[k % 2],
                                [jnp.full((16,), d, jnp.int32), col])
                            stageT[pl.ds(d * 16, 16)] = x_d
                        for e in range(16):
                            ev = jnp.full((16,), e, jnp.int32)
                            for h in range(2):
                                dvec = lanes + 16 * h
                                x = plsc.load_gather(
                                    stageT, [dvec * 16 + ev])
                                plsc.store_scatter(
                                    rows_st, [ev, dvec], x)
                        sidx = jnp.where(m, pos, -1)
                        pltpu.async_copy(
                            rows_st,
                            out_hbm.at[plsc.Indices(
                                sidx, ignored_value=-1)],
                            sem2).wait()
                    return carry2

                lax.fori_loop(0, nch, ch_body, 0)
            return carry

        lax.fori_loop(0, KMAX, k_body, 0)


def _dot_body(eu_hbm, ei_hbm, wb_hbm, out_hbm,
              bu, bv, wb_v, out_v, sem):
    c = lax.axis_index("c")
    s_ax = lax.axis_index("s")
    wid = s_ax * NC + c
    lanes = lax.iota(jnp.int32, 16)

    pltpu.sync_copy(wb_hbm, wb_v)
    ws = [plsc.load_gather(wb_v, [jnp.full((16,), d, jnp.int32)])
          for d in range(MF_DIM)]
    bv_b = plsc.load_gather(wb_v, [jnp.full((16,), MF_DIM, jnp.int32)])

    def fire(p):
        base = wid * B_PER_W + p * 128
        pltpu.async_copy(eu_hbm.at[pl.ds(base, 128), :], bu.at[p % 2], sem)
        pltpu.async_copy(ei_hbm.at[pl.ds(base, 128), :], bv.at[p % 2], sem)

    fire(0)
    fire(1)
    for p in range(4):
        pltpu.make_async_copy(
            eu_hbm.at[pl.ds(0, 128), :], bu.at[p % 2], sem).wait()
        pltpu.make_async_copy(
            ei_hbm.at[pl.ds(0, 128), :], bv.at[p % 2], sem).wait()
        if p + 2 < 4:
            fire(p + 2)

        def g_body(g, carry, p=p):
            rows = g * 16 + lanes
            acc = bv_b
            for d in range(MF_DIM):
                dcol = jnp.full((16,), d, jnp.int32)
                u_d = plsc.load_gather(bu.at[p % 2], [rows, dcol])
                v_d = plsc.load_gather(bv.at[p % 2], [rows, dcol])
                acc = acc + u_d * v_d * ws[d]
            out_v[pl.ds(p * 128 + g * 16, 16)] = (
                1.0 / (1.0 + jnp.exp(-acc)))
            return carry

        lax.fori_loop(0, 8, g_body, 0)

    pltpu.sync_copy(out_v, out_hbm.at[pl.ds(wid * B_PER_W, B_PER_W)])


_MESH = plsc.VectorSubcoreMesh(core_axis_name="c", subcore_axis_name="s")
_PARAMS = pltpu.CompilerParams(
    needs_layout_passes=False, use_tc_tiling_on_sc=True)


@functools.partial(
    pl.kernel, mesh=_MESH, compiler_params=_PARAMS,
    out_type=(jax.ShapeDtypeStruct((BATCH, 128), jnp.float32),
              jax.ShapeDtypeStruct((BATCH, 128), jnp.float32)),
    scratch_types=[
        pltpu.VMEM((BATCH,), jnp.int32),
        pltpu.VMEM((BATCH + 16,), jnp.int32),
        pltpu.VMEM((2, MF_DIM, SW), jnp.float32),
        pltpu.VMEM((MF_DIM * 16,), jnp.float32),
        pltpu.VMEM((16, 128), jnp.float32),
        pltpu.SemaphoreType.DMA,
        pltpu.SemaphoreType.DMA,
    ],
)
def _extract_sc(*args):
    _extract_body(*args)


@functools.partial(
    pl.kernel, mesh=_MESH, compiler_params=_PARAMS,
    out_type=jax.ShapeDtypeStruct((BATCH,), jnp.float32),
    scratch_types=[
        pltpu.VMEM((2, 128, 128), jnp.float32),
        pltpu.VMEM((2, 128, 128), jnp.float32),
        pltpu.VMEM((48,), jnp.float32),
        pltpu.VMEM((B_PER_W,), jnp.float32),
        pltpu.SemaphoreType.DMA,
    ],
)
def _dot_sc(*args):
    _dot_body(*args)


def kernel(user_input, item_input, user_table, item_table, W, b):
    ui = user_input.astype(jnp.int32)
    ii = item_input.astype(jnp.int32)
    wb = jnp.concatenate([
        W.reshape(MF_DIM).astype(jnp.float32),
        b.reshape(1).astype(jnp.float32),
        jnp.zeros((15,), jnp.float32),
    ])
    eu, ei = _extract_sc(ui, ii, user_table.T, item_table.T)
    out = _dot_sc(eu, ei, wb)
    return out.reshape(BATCH, 1)


# speedup vs baseline: 2.4656x; 2.4656x over previous
import functools

import jax
import jax.numpy as jnp
from jax import lax
from jax.experimental import pallas as pl
from jax.experimental.pallas import tpu as pltpu
from jax.experimental.pallas import tpu_sc as plsc

MF_DIM = 32
BATCH = 16384
NC = 2
NS = 16
NW = NC * NS
B_PER_W = BATCH // NW
BLK = 128
WAVE = 16


def _gmf_body(ui_hbm, ii_hbm, ut_hbm, it_hbm, wb_hbm, out_hbm,
              idx_uv, idx_iv, blks, cu, ci, wb_v, out_v, sem):
    c = lax.axis_index("c")
    s = lax.axis_index("s")
    wid = s * NC + c

    pltpu.sync_copy(ui_hbm.at[wid], idx_uv)
    pltpu.sync_copy(ii_hbm.at[wid], idx_iv)
    pltpu.sync_copy(wb_hbm, wb_v)

    lanes = lax.iota(jnp.int32, 16)

    def wave(step, idx_ref, tab_hbm, dst):
        chunk = idx_ref[pl.ds(step * WAVE, WAVE)]
        rs = [chunk[lane] for lane in range(WAVE)]
        for lane in range(WAVE):
            base = pl.multiple_of((rs[lane] >> 7) * BLK, BLK)
            pltpu.async_copy(
                tab_hbm.at[:, pl.ds(base, BLK)], blks.at[lane], sem)
        for lane in range(WAVE):
            pltpu.make_async_copy(
                tab_hbm.at[:, pl.ds(0, BLK)], blks.at[lane], sem).wait()
        evec = step * WAVE + lanes
        for lane in range(WAVE):
            col = jnp.full((16,), rs[lane] & (BLK - 1), jnp.int32)
            ev = jnp.full((16,), step * WAVE + lane, jnp.int32)
            for h in range(2):
                dvec = lanes + 16 * h
                x = plsc.load_gather(blks.at[lane], [dvec, col])
                plsc.store_scatter(dst, [dvec, ev], x)
        del evec

    def w_body(step, carry):
        wave(step, idx_uv, ut_hbm, cu)
        wave(step, idx_iv, it_hbm, ci)
        return carry

    lax.fori_loop(0, B_PER_W // WAVE, w_body, 0)

    ws = [plsc.load_gather(wb_v, [jnp.full((16,), d, jnp.int32)])
          for d in range(MF_DIM)]
    bv = plsc.load_gather(wb_v, [jnp.full((16,), MF_DIM, jnp.int32)])

    def g_body(g, carry):
        rows = g * 16 + lanes
        acc = bv
        for d in range(MF_DIM):
            dcol = jnp.full((16,), d, jnp.int32)
            u_d = plsc.load_gather(cu, [dcol, rows])
            v_d = plsc.load_gather(ci, [dcol, rows])
            acc = acc + u_d * v_d * ws[d]
        out_v[pl.ds(g * 16, 16)] = 1.0 / (1.0 + jnp.exp(-acc))
        return carry

    lax.fori_loop(0, B_PER_W // 16, g_body, 0)
    pltpu.sync_copy(out_v, out_hbm.at[pl.ds(wid * B_PER_W, B_PER_W)])


@functools.partial(
    pl.kernel,
    mesh=plsc.VectorSubcoreMesh(core_axis_name="c", subcore_axis_name="s"),
    out_type=jax.ShapeDtypeStruct((BATCH,), jnp.float32),
    compiler_params=pltpu.CompilerParams(
        needs_layout_passes=False, use_tc_tiling_on_sc=True),
    scratch_types=[
        pltpu.VMEM((B_PER_W,), jnp.int32),
        pltpu.VMEM((B_PER_W,), jnp.int32),
        pltpu.VMEM((WAVE, MF_DIM, BLK), jnp.float32),
        pltpu.VMEM((MF_DIM, B_PER_W), jnp.float32),
        pltpu.VMEM((MF_DIM, B_PER_W), jnp.float32),
        pltpu.VMEM((48,), jnp.float32),
        pltpu.VMEM((B_PER_W,), jnp.float32),
        pltpu.SemaphoreType.DMA,
    ],
)
def _gmf_sc(*args):
    _gmf_body(*args)


def kernel(user_input, item_input, user_table, item_table, W, b):
    ui = user_input.astype(jnp.int32).reshape(NW, B_PER_W)
    ii = item_input.astype(jnp.int32).reshape(NW, B_PER_W)
    wb = jnp.concatenate([
        W.reshape(MF_DIM).astype(jnp.float32),
        b.reshape(1).astype(jnp.float32),
        jnp.zeros((15,), jnp.float32),
    ])
    out = _gmf_sc(ui, ii, user_table.T, item_table.T, wb)
    return out.reshape(BATCH, 1)
